# Initial kernel scaffold; baseline (speedup 1.0000x reference)
import jax
import jax.numpy as jnp
from jax.experimental import pallas as pl


def kernel(inputs, mean, cvar_inv):
    raise NotImplementedError("write your pallas kernel here")


# speedup vs baseline: 1.6291x; 1.6291x over previous
import numpy as np
import jax
import jax.numpy as jnp
from jax.experimental import pallas as pl

B, H, W, C = 4, 32, 32, 192
N = H * W
IMG = 512
SIGMA = 4.0
KS = 33
_NB = 64

_HIGH = jax.lax.Precision.HIGHEST


def _resize_blur_matrix():
    out_size, in_size = IMG, H
    s = (np.arange(out_size, dtype=np.float64) + 0.5) * (in_size / out_size) - 0.5
    j = np.arange(in_size, dtype=np.float64)
    w = np.maximum(0.0, 1.0 - np.abs(s[None, :] - j[:, None]))
    w /= w.sum(axis=0, keepdims=True)
    R = w.T
    x = np.arange(KS, dtype=np.float64) - (KS - 1) / 2.0
    g = np.exp(-(x ** 2) / (2.0 * SIGMA * SIGMA))
    g /= g.sum()
    pad = KS // 2
    Bl = np.zeros((out_size, out_size), dtype=np.float64)
    for i in range(out_size):
        for t in range(KS):
            src = i - pad + t
            if src < 0:
                src = -src
            elif src >= out_size:
                src = 2 * out_size - 2 - src
            Bl[i, src] += g[t]
    return (Bl @ R).astype(np.float32)


_A = _resize_blur_matrix()


def _maha_kernel(f_ref, m_ref, c_ref, o_ref):
    def body(i, carry):
        x = f_ref[i] - m_ref[i]
        y = jax.lax.dot(x, c_ref[i], precision=_HIGH)
        d2 = jnp.sum(y * x, axis=1)
        o_ref[i, :] = jnp.sqrt(jnp.maximum(d2, 0.0))
        return carry

    jax.lax.fori_loop(0, _NB, body, 0)


def _mask_kernel(d_ref, a_ref, at_ref, mask_ref, score_ref):
    a = a_ref[...]
    at = at_ref[...]
    scores = []
    for b in range(B):
        t = jax.lax.dot(a, d_ref[b], precision=_HIGH)
        m = jax.lax.dot(t, at, precision=_HIGH)
        mask_ref[b] = m
        scores.append(jnp.max(m))
    score_ref[...] = jnp.stack(scores).reshape(B, 1)


def kernel(inputs, mean, cvar_inv):
    feature = inputs.reshape(B, N, C).transpose(1, 0, 2)
    mean3 = mean.reshape(N, 1, C)

    dist_nb = pl.pallas_call(
        _maha_kernel,
        grid=(N // _NB,),
        in_specs=[
            pl.BlockSpec((_NB, B, C), lambda i: (i, 0, 0)),
            pl.BlockSpec((_NB, 1, C), lambda i: (i, 0, 0)),
            pl.BlockSpec((_NB, C, C), lambda i: (i, 0, 0)),
        ],
        out_specs=pl.BlockSpec((_NB, B), lambda i: (i, 0)),
        out_shape=jax.ShapeDtypeStruct((N, B), jnp.float32),
    )(feature, mean3, cvar_inv)

    dist = dist_nb.T.reshape(B, H, W)
    a = jnp.asarray(_A)
    mask, score = pl.pallas_call(
        _mask_kernel,
        out_shape=[
            jax.ShapeDtypeStruct((B, IMG, IMG), jnp.float32),
            jax.ShapeDtypeStruct((B, 1), jnp.float32),
        ],
    )(dist, a, a.T)

    return (score, mask.reshape(B, IMG, IMG, 1))


# speedup vs baseline: 3.0271x; 1.8581x over previous
import numpy as np
import jax
import jax.numpy as jnp
from jax.experimental import pallas as pl

B, H, W, C = 4, 32, 32, 192
N = H * W
IMG = 512
SIGMA = 4.0
KS = 33
_NB = 64

_HIGH = jax.lax.Precision.HIGHEST


def _resize_blur_matrix():
    out_size, in_size = IMG, H
    s = (np.arange(out_size, dtype=np.float64) + 0.5) * (in_size / out_size) - 0.5
    j = np.arange(in_size, dtype=np.float64)
    w = np.maximum(0.0, 1.0 - np.abs(s[None, :] - j[:, None]))
    w /= w.sum(axis=0, keepdims=True)
    R = w.T
    x = np.arange(KS, dtype=np.float64) - (KS - 1) / 2.0
    g = np.exp(-(x ** 2) / (2.0 * SIGMA * SIGMA))
    g /= g.sum()
    pad = KS // 2
    Bl = np.zeros((out_size, out_size), dtype=np.float64)
    for i in range(out_size):
        for t in range(KS):
            src = i - pad + t
            if src < 0:
                src = -src
            elif src >= out_size:
                src = 2 * out_size - 2 - src
            Bl[i, src] += g[t]
    return (Bl @ R).astype(np.float32)


_A = _resize_blur_matrix()


def _maha_kernel(f_ref, m_ref, c_ref, o_ref):
    delta = f_ref[...] - m_ref[...]
    y = jax.lax.dot_general(
        delta, c_ref[...],
        dimension_numbers=(((2,), (1,)), ((0,), (0,))),
        precision=jax.lax.Precision.DEFAULT,
    )
    d2 = jnp.sum(y * delta, axis=2)
    o_ref[...] = jnp.sqrt(jnp.maximum(d2, 0.0))


def _mask_kernel(d_ref, a_ref, at_ref, mask_ref, score_ref):
    a = a_ref[...]
    at = at_ref[...]
    scores = []
    for b in range(B):
        t = jax.lax.dot(a, d_ref[b], precision=_HIGH)
        m = jax.lax.dot(t, at, precision=_HIGH)
        mask_ref[b] = m
        scores.append(jnp.max(m))
    score_ref[...] = jnp.stack(scores).reshape(B, 1)


def kernel(inputs, mean, cvar_inv):
    feature = inputs.reshape(B, N, C).transpose(1, 0, 2)
    mean3 = mean.reshape(N, 1, C)

    dist_nb = pl.pallas_call(
        _maha_kernel,
        grid=(N // _NB,),
        in_specs=[
            pl.BlockSpec((_NB, B, C), lambda i: (i, 0, 0)),
            pl.BlockSpec((_NB, 1, C), lambda i: (i, 0, 0)),
            pl.BlockSpec((_NB, C, C), lambda i: (i, 0, 0)),
        ],
        out_specs=pl.BlockSpec((_NB, B), lambda i: (i, 0)),
        out_shape=jax.ShapeDtypeStruct((N, B), jnp.float32),
    )(feature, mean3, cvar_inv)

    dist = dist_nb.T.reshape(B, H, W)
    a = jnp.asarray(_A)
    mask, score = pl.pallas_call(
        _mask_kernel,
        out_shape=[
            jax.ShapeDtypeStruct((B, IMG, IMG), jnp.float32),
            jax.ShapeDtypeStruct((B, 1), jnp.float32),
        ],
    )(dist, a, a.T)

    return (score, mask.reshape(B, IMG, IMG, 1))
